# Initial kernel scaffold; baseline (speedup 1.0000x reference)
import jax
import jax.numpy as jnp
from jax.experimental import pallas as pl


def kernel(x, edge_index, W1, b1, W2, b2):
    raise NotImplementedError("write your pallas kernel here")


# speedup vs baseline: 27.2671x; 27.2671x over previous
import functools

import jax
import jax.numpy as jnp
from jax import lax
from jax.experimental import pallas as pl
from jax.experimental.pallas import tpu as pltpu
from jax.experimental.pallas import tpu_sc as plsc

NC = 2
NS = 16
NW = NC * NS
K = 128


def _zero_rows(ref, nrows, ncols):
    z = jnp.zeros((16,), jnp.float32)

    def body(i, c):
        for k4 in range(ncols // 16):
            ref[i, pl.ds(16 * k4, 16)] = z
        return c

    lax.fori_loop(0, nrows, body, 0, unroll=4)


def _fill_ones(ref, nrows):
    o = jnp.ones((16,), jnp.float32)

    def body(i, c):
        ref[i, :] = o
        return c

    lax.fori_loop(0, nrows, body, 0, unroll=4)


def _zero_acc_slice(zsrc, acc, base, rpt):
    n_full = rpt // K
    rem = rpt - n_full * K

    def body(i, c):
        pltpu.sync_copy(zsrc, acc.at[pl.ds(base + i * K, K)])
        return c

    lax.fori_loop(0, n_full, body, 0)
    if rem:
        pltpu.sync_copy(zsrc.at[pl.ds(0, rem)],
                        acc.at[pl.ds(base + n_full * K, rem)])


def _make_deg_kernel(n_acc, cpw):
    rpt = n_acc // NS

    @functools.partial(
        pl.kernel,
        out_type=jax.ShapeDtypeStruct((NC, n_acc, 16), jnp.float32),
        mesh=plsc.VectorSubcoreMesh(core_axis_name="c", subcore_axis_name="s"),
        scratch_types=[
            pltpu.VMEM((cpw, K), jnp.int32),
            pltpu.VMEM((K, 16), jnp.float32),
            pltpu.VMEM((K, 16), jnp.float32),
            pltpu.VMEM_SHARED((n_acc, 16), jnp.float32),
        ],
        compiler_params=pltpu.CompilerParams(use_tc_tiling_on_sc=False),
    )
    def deg_kernel(dst_hbm, out_hbm, didx, ones_b, zero_b, acc):
        c = lax.axis_index("c")
        s = lax.axis_index("s")
        wid = s * NC + c
        base = s * rpt
        _fill_ones(ones_b, K)
        _zero_rows(zero_b, K, 16)
        _zero_acc_slice(zero_b, acc, base, rpt)
        pltpu.sync_copy(dst_hbm.at[pl.ds(wid * cpw, cpw)], didx)
        plsc.subcore_barrier()

        def step(j, carry):
            pltpu.sync_copy(ones_b, acc.at[didx.at[j]], add=True)
            return carry

        lax.fori_loop(0, cpw, step, 0)
        plsc.subcore_barrier()
        pltpu.sync_copy(acc.at[pl.ds(base, rpt)],
                        out_hbm.at[c, pl.ds(base, rpt)])

    return deg_kernel


def _make_agg_kernel(n, n_acc, d, cpw):
    rpt = n_acc // NS

    @functools.partial(
        pl.kernel,
        out_type=jax.ShapeDtypeStruct((NC, n_acc, d), jnp.float32),
        mesh=plsc.VectorSubcoreMesh(core_axis_name="c", subcore_axis_name="s"),
        scratch_types=[
            pltpu.VMEM((cpw, K), jnp.int32),
            pltpu.VMEM((cpw, K), jnp.int32),
            pltpu.VMEM((K, d), jnp.float32),
            pltpu.VMEM((K, d), jnp.float32),
            pltpu.VMEM_SHARED((n_acc, d), jnp.float32),
            pltpu.SemaphoreType.DMA,
        ],
        compiler_params=pltpu.CompilerParams(use_tc_tiling_on_sc=False),
    )
    def agg_kernel(h_hbm, src_hbm, dst_hbm, out_hbm,
                   sidx, didx, rows0, rows1, acc, sem):
        c = lax.axis_index("c")
        s = lax.axis_index("s")
        wid = s * NC + c
        base = s * rpt
        _zero_rows(rows0, K, d)
        _zero_acc_slice(rows0, acc, base, rpt)
        pltpu.sync_copy(src_hbm.at[pl.ds(wid * cpw, cpw)], sidx)
        pltpu.sync_copy(dst_hbm.at[pl.ds(wid * cpw, cpw)], didx)
        plsc.subcore_barrier()

        del rows1

        def step(j, carry):
            pltpu.async_copy(h_hbm.at[sidx.at[j]], rows0, sem).wait()
            pltpu.sync_copy(rows0, acc.at[didx.at[j]], add=True)
            return carry

        lax.fori_loop(0, cpw, step, 0)
        plsc.subcore_barrier()
        pltpu.sync_copy(acc.at[pl.ds(base, rpt)],
                        out_hbm.at[c, pl.ds(base, rpt)])

    return agg_kernel


def _matmul_call(x, w, bm):
    n, din = x.shape
    dout = w.shape[1]

    def body(x_ref, w_ref, o_ref):
        o_ref[...] = jnp.dot(x_ref[...], w_ref[...],
                             preferred_element_type=jnp.float32)

    return pl.pallas_call(
        body,
        grid=(n // bm,),
        in_specs=[pl.BlockSpec((bm, din), lambda i: (i, 0)),
                  pl.BlockSpec((din, dout), lambda i: (0, 0))],
        out_specs=pl.BlockSpec((bm, dout), lambda i: (i, 0)),
        out_shape=jax.ShapeDtypeStruct((n, dout), jnp.float32),
    )(x, w)


def _scale1_call(h1, d0, d1, bm):
    n, d = h1.shape

    def body(h_ref, d0_ref, d1_ref, hs_ref, dis_ref):
        deg = d0_ref[...][:, 0:1] + d1_ref[...][:, 0:1] + 1.0
        dis = lax.rsqrt(deg)
        dis_ref[...] = dis
        hs_ref[...] = h_ref[...] * dis

    return pl.pallas_call(
        body,
        grid=(n // bm,),
        in_specs=[pl.BlockSpec((bm, d), lambda i: (i, 0)),
                  pl.BlockSpec((bm, 16), lambda i: (i, 0)),
                  pl.BlockSpec((bm, 16), lambda i: (i, 0))],
        out_specs=[pl.BlockSpec((bm, d), lambda i: (i, 0)),
                   pl.BlockSpec((bm, 1), lambda i: (i, 0))],
        out_shape=[jax.ShapeDtypeStruct((n, d), jnp.float32),
                   jax.ShapeDtypeStruct((n, 1), jnp.float32)],
    )(h1, d0, d1)


def _mid_call(p0, p1, h1s, dis, b1, bm):
    n, d = h1s.shape

    def body(p0_ref, p1_ref, h_ref, dis_ref, b_ref, g_ref):
        t = (p0_ref[...] + p1_ref[...] + h_ref[...]) * dis_ref[...] + b_ref[...]
        g_ref[...] = jnp.maximum(t, 0.0) * dis_ref[...]

    return pl.pallas_call(
        body,
        grid=(n // bm,),
        in_specs=[pl.BlockSpec((bm, d), lambda i: (i, 0)),
                  pl.BlockSpec((bm, d), lambda i: (i, 0)),
                  pl.BlockSpec((bm, d), lambda i: (i, 0)),
                  pl.BlockSpec((bm, 1), lambda i: (i, 0)),
                  pl.BlockSpec((1, d), lambda i: (0, 0))],
        out_specs=pl.BlockSpec((bm, d), lambda i: (i, 0)),
        out_shape=jax.ShapeDtypeStruct((n, d), jnp.float32),
    )(p0, p1, h1s, dis, b1)


def _out_call(q0, q1, g, dis, w2, b2, bm):
    n, d = g.shape
    dout = w2.shape[1]

    def body(q0_ref, q1_ref, g_ref, dis_ref, w_ref, b_ref, o_ref):
        u = (q0_ref[...] + q1_ref[...] + g_ref[...]) * dis_ref[...]
        o_ref[...] = jnp.dot(u, w_ref[...],
                             preferred_element_type=jnp.float32) + b_ref[...]

    return pl.pallas_call(
        body,
        grid=(n // bm,),
        in_specs=[pl.BlockSpec((bm, d), lambda i: (i, 0)),
                  pl.BlockSpec((bm, d), lambda i: (i, 0)),
                  pl.BlockSpec((bm, d), lambda i: (i, 0)),
                  pl.BlockSpec((bm, 1), lambda i: (i, 0)),
                  pl.BlockSpec((d, dout), lambda i: (0, 0)),
                  pl.BlockSpec((1, dout), lambda i: (0, 0))],
        out_specs=pl.BlockSpec((bm, dout), lambda i: (i, 0)),
        out_shape=jax.ShapeDtypeStruct((n, dout), jnp.float32),
    )(q0, q1, g, dis, w2, b2)


def kernel(x, edge_index, W1, b1, W2, b2):
    n, d_in = x.shape
    d_hid = W1.shape[1]
    d_out = W2.shape[1]
    e = edge_index.shape[1]
    n_acc = -(-(n + 1) // (NS * 8)) * (NS * 8)
    trash = n_acc - n
    cpw = -(-e // (NW * K * 8)) * 8
    e_pad = NW * cpw * K
    pad = e_pad - e

    src = edge_index[0]
    dst = edge_index[1]
    ar = jnp.arange(pad, dtype=jnp.int32)
    src_p = jnp.concatenate([src, (ar * 89) % n]).reshape(NW * cpw, K)
    dst_p = jnp.concatenate([dst, n + (ar % trash)]).reshape(NW * cpw, K)

    bm = 2000 if n % 2000 == 0 else n

    degp = _make_deg_kernel(n_acc, cpw)(dst_p)
    h1 = _matmul_call(x, W1, bm)
    h1s, dis = _scale1_call(h1, degp[0, :n], degp[1, :n], bm)

    agg = _make_agg_kernel(n, n_acc, d_hid, cpw)
    p = agg(h1s, src_p, dst_p)
    g = _mid_call(p[0, :n], p[1, :n], h1s, dis, b1.reshape(1, d_hid), bm)
    q = agg(g, src_p, dst_p)
    return _out_call(q[0, :n], q[1, :n], g, dis, W2, b2.reshape(1, d_out), bm)


# speedup vs baseline: 37.3783x; 1.3708x over previous
import functools

import jax
import jax.numpy as jnp
from jax import lax
from jax.experimental import pallas as pl
from jax.experimental.pallas import tpu as pltpu
from jax.experimental.pallas import tpu_sc as plsc

NC = 2
NS = 16
NW = NC * NS
K = 128


def _zero_rows(ref, nrows, ncols):
    z = jnp.zeros((16,), jnp.float32)

    def body(i, c):
        for k4 in range(ncols // 16):
            ref[i, pl.ds(16 * k4, 16)] = z
        return c

    lax.fori_loop(0, nrows, body, 0, unroll=4)


def _fill_ones(ref, nrows):
    o = jnp.ones((16,), jnp.float32)

    def body(i, c):
        ref[i, :] = o
        return c

    lax.fori_loop(0, nrows, body, 0, unroll=4)


def _zero_acc_slice(zsrc, acc, base, rpt):
    n_full = rpt // K
    rem = rpt - n_full * K

    def body(i, c):
        pltpu.sync_copy(zsrc, acc.at[pl.ds(base + i * K, K)])
        return c

    lax.fori_loop(0, n_full, body, 0)
    if rem:
        pltpu.sync_copy(zsrc.at[pl.ds(0, rem)],
                        acc.at[pl.ds(base + n_full * K, rem)])


def _make_deg_kernel(n_acc, cpw):
    rpt = n_acc // NS

    @functools.partial(
        pl.kernel,
        out_type=jax.ShapeDtypeStruct((NC, n_acc, 16), jnp.float32),
        mesh=plsc.VectorSubcoreMesh(core_axis_name="c", subcore_axis_name="s"),
        scratch_types=[
            pltpu.VMEM((cpw, K), jnp.int32),
            pltpu.VMEM((K, 16), jnp.float32),
            pltpu.VMEM((K, 16), jnp.float32),
            pltpu.VMEM_SHARED((n_acc, 16), jnp.float32),
        ],
        compiler_params=pltpu.CompilerParams(use_tc_tiling_on_sc=False),
    )
    def deg_kernel(dst_hbm, out_hbm, didx, ones_b, zero_b, acc):
        c = lax.axis_index("c")
        s = lax.axis_index("s")
        wid = s * NC + c
        base = s * rpt
        _fill_ones(ones_b, K)
        _zero_rows(zero_b, K, 16)
        _zero_acc_slice(zero_b, acc, base, rpt)
        pltpu.sync_copy(dst_hbm.at[pl.ds(wid * cpw, cpw)], didx)
        plsc.subcore_barrier()

        def step(j, carry):
            pltpu.sync_copy(ones_b, acc.at[didx.at[j]], add=True)
            return carry

        lax.fori_loop(0, cpw, step, 0)
        plsc.subcore_barrier()
        pltpu.sync_copy(acc.at[pl.ds(base, rpt)],
                        out_hbm.at[c, pl.ds(base, rpt)])

    return deg_kernel


def _make_agg_kernel(n, n_acc, d, cpw):
    rpt = n_acc // NS

    @functools.partial(
        pl.kernel,
        out_type=jax.ShapeDtypeStruct((NC, n_acc, d), jnp.float32),
        mesh=plsc.VectorSubcoreMesh(core_axis_name="c", subcore_axis_name="s"),
        scratch_types=[
            pltpu.VMEM((cpw, K), jnp.int32),
            pltpu.VMEM((cpw, K), jnp.int32),
            pltpu.VMEM((K, d), jnp.float32),
            pltpu.VMEM((K, d), jnp.float32),
            pltpu.VMEM_SHARED((n_acc, d), jnp.float32),
            pltpu.SemaphoreType.DMA,
            pltpu.SemaphoreType.DMA,
        ],
        compiler_params=pltpu.CompilerParams(use_tc_tiling_on_sc=False),
    )
    def agg_kernel(h_hbm, src_hbm, dst_hbm, out_hbm,
                   sidx, didx, rows0, rows1, acc, sem0, sem1):
        c = lax.axis_index("c")
        s = lax.axis_index("s")
        wid = s * NC + c
        base = s * rpt
        _zero_rows(rows0, K, d)
        _zero_acc_slice(rows0, acc, base, rpt)
        pltpu.sync_copy(src_hbm.at[pl.ds(wid * cpw, cpw)], sidx)
        pltpu.sync_copy(dst_hbm.at[pl.ds(wid * cpw, cpw)], didx)
        plsc.subcore_barrier()

        pltpu.async_copy(h_hbm.at[sidx.at[0]], rows0, sem0)

        def step(i, carry):
            j = 2 * i
            pltpu.async_copy(h_hbm.at[sidx.at[j + 1]], rows1, sem1)
            pltpu.make_async_copy(h_hbm.at[sidx.at[j]], rows0, sem0).wait()
            pltpu.sync_copy(rows0, acc.at[didx.at[j]], add=True)

            @pl.when(j + 2 < cpw)
            def _():
                pltpu.async_copy(h_hbm.at[sidx.at[j + 2]], rows0, sem0)

            pltpu.make_async_copy(h_hbm.at[sidx.at[j + 1]], rows1, sem1,
                                  ).wait()
            pltpu.sync_copy(rows1, acc.at[didx.at[j + 1]], add=True)
            return carry

        assert cpw % 2 == 0
        lax.fori_loop(0, cpw // 2, step, 0)
        plsc.subcore_barrier()
        pltpu.sync_copy(acc.at[pl.ds(base, rpt)],
                        out_hbm.at[c, pl.ds(base, rpt)])

    return agg_kernel


def _matmul_call(x, w, bm):
    n, din = x.shape
    dout = w.shape[1]

    def body(x_ref, w_ref, o_ref):
        o_ref[...] = jnp.dot(x_ref[...], w_ref[...],
                             preferred_element_type=jnp.float32)

    return pl.pallas_call(
        body,
        grid=(n // bm,),
        in_specs=[pl.BlockSpec((bm, din), lambda i: (i, 0)),
                  pl.BlockSpec((din, dout), lambda i: (0, 0))],
        out_specs=pl.BlockSpec((bm, dout), lambda i: (i, 0)),
        out_shape=jax.ShapeDtypeStruct((n, dout), jnp.float32),
    )(x, w)


def _scale1_call(h1, d0, d1, bm):
    n, d = h1.shape

    def body(h_ref, d0_ref, d1_ref, hs_ref, dis_ref):
        deg = d0_ref[...][:, 0:1] + d1_ref[...][:, 0:1] + 1.0
        dis = lax.rsqrt(deg)
        dis_ref[...] = dis
        hs_ref[...] = h_ref[...] * dis

    return pl.pallas_call(
        body,
        grid=(n // bm,),
        in_specs=[pl.BlockSpec((bm, d), lambda i: (i, 0)),
                  pl.BlockSpec((bm, 16), lambda i: (i, 0)),
                  pl.BlockSpec((bm, 16), lambda i: (i, 0))],
        out_specs=[pl.BlockSpec((bm, d), lambda i: (i, 0)),
                   pl.BlockSpec((bm, 1), lambda i: (i, 0))],
        out_shape=[jax.ShapeDtypeStruct((n, d), jnp.float32),
                   jax.ShapeDtypeStruct((n, 1), jnp.float32)],
    )(h1, d0, d1)


def _mid_call(p0, p1, h1s, dis, b1, bm):
    n, d = h1s.shape

    def body(p0_ref, p1_ref, h_ref, dis_ref, b_ref, g_ref):
        t = (p0_ref[...] + p1_ref[...] + h_ref[...]) * dis_ref[...] + b_ref[...]
        g_ref[...] = jnp.maximum(t, 0.0) * dis_ref[...]

    return pl.pallas_call(
        body,
        grid=(n // bm,),
        in_specs=[pl.BlockSpec((bm, d), lambda i: (i, 0)),
                  pl.BlockSpec((bm, d), lambda i: (i, 0)),
                  pl.BlockSpec((bm, d), lambda i: (i, 0)),
                  pl.BlockSpec((bm, 1), lambda i: (i, 0)),
                  pl.BlockSpec((1, d), lambda i: (0, 0))],
        out_specs=pl.BlockSpec((bm, d), lambda i: (i, 0)),
        out_shape=jax.ShapeDtypeStruct((n, d), jnp.float32),
    )(p0, p1, h1s, dis, b1)


def _out_call(q0, q1, g, dis, w2, b2, bm):
    n, d = g.shape
    dout = w2.shape[1]

    def body(q0_ref, q1_ref, g_ref, dis_ref, w_ref, b_ref, o_ref):
        u = (q0_ref[...] + q1_ref[...] + g_ref[...]) * dis_ref[...]
        o_ref[...] = jnp.dot(u, w_ref[...],
                             preferred_element_type=jnp.float32) + b_ref[...]

    return pl.pallas_call(
        body,
        grid=(n // bm,),
        in_specs=[pl.BlockSpec((bm, d), lambda i: (i, 0)),
                  pl.BlockSpec((bm, d), lambda i: (i, 0)),
                  pl.BlockSpec((bm, d), lambda i: (i, 0)),
                  pl.BlockSpec((bm, 1), lambda i: (i, 0)),
                  pl.BlockSpec((d, dout), lambda i: (0, 0)),
                  pl.BlockSpec((1, dout), lambda i: (0, 0))],
        out_specs=pl.BlockSpec((bm, dout), lambda i: (i, 0)),
        out_shape=jax.ShapeDtypeStruct((n, dout), jnp.float32),
    )(q0, q1, g, dis, w2, b2)


def kernel(x, edge_index, W1, b1, W2, b2):
    n, d_in = x.shape
    d_hid = W1.shape[1]
    d_out = W2.shape[1]
    e = edge_index.shape[1]
    n_acc = -(-(n + 1) // (NS * 8)) * (NS * 8)
    trash = n_acc - n
    cpw = -(-e // (NW * K * 8)) * 8
    e_pad = NW * cpw * K
    pad = e_pad - e

    src = edge_index[0]
    dst = edge_index[1]
    ar = jnp.arange(pad, dtype=jnp.int32)
    src_p = jnp.concatenate([src, (ar * 89) % n]).reshape(NW * cpw, K)
    dst_p = jnp.concatenate([dst, n + (ar % trash)]).reshape(NW * cpw, K)

    bm = 2000 if n % 2000 == 0 else n

    degp = _make_deg_kernel(n_acc, cpw)(dst_p)
    h1 = _matmul_call(x, W1, bm)
    h1s, dis = _scale1_call(h1, degp[0, :n], degp[1, :n], bm)

    agg = _make_agg_kernel(n, n_acc, d_hid, cpw)
    p = agg(h1s, src_p, dst_p)
    g = _mid_call(p[0, :n], p[1, :n], h1s, dis, b1.reshape(1, d_hid), bm)
    q = agg(g, src_p, dst_p)
    return _out_call(q[0, :n], q[1, :n], g, dis, W2, b2.reshape(1, d_out), bm)
